# baseline (speedup 1.0000x reference)
import jax
import jax.numpy as jnp
from jax.experimental import pallas as pl

BATCH = 16384
NUM_CLASSES = 1000
CB = 8


def _body(x_ref, o_ref):
    c0 = pl.program_id(0) * CB
    cls = jax.lax.broadcasted_iota(jnp.int32, (CB, BATCH), 0) + c0
    o_ref[...] = (cls == x_ref[...][None, :]).astype(jnp.float32)


def kernel(x):
    out_t = pl.pallas_call(
        _body,
        grid=(NUM_CLASSES // CB,),
        in_specs=[pl.BlockSpec((BATCH,), lambda i: (0,))],
        out_specs=pl.BlockSpec((CB, BATCH), lambda i: (i, 0)),
        out_shape=jax.ShapeDtypeStruct((NUM_CLASSES, BATCH), jnp.float32),
    )(x)
    return out_t.T


# speedup vs baseline: 2.4045x; 2.4045x over previous
import jax
import jax.numpy as jnp
from jax.experimental import pallas as pl

BATCH = 16384
NUM_CLASSES = 1000
CB = 40


def _body(x_ref, o_ref):
    c0 = pl.program_id(0) * CB
    cls = jax.lax.broadcasted_iota(jnp.int32, (CB, BATCH), 0) + c0
    o_ref[...] = (cls == x_ref[...][None, :]).astype(jnp.float32)


def kernel(x):
    out_t = pl.pallas_call(
        _body,
        grid=(NUM_CLASSES // CB,),
        in_specs=[pl.BlockSpec((BATCH,), lambda i: (0,))],
        out_specs=pl.BlockSpec((CB, BATCH), lambda i: (i, 0)),
        out_shape=jax.ShapeDtypeStruct((NUM_CLASSES, BATCH), jnp.float32),
    )(x)
    return out_t.T
